# baseline (speedup 1.0000x reference)
import functools

import jax
import jax.numpy as jnp
from jax import lax
from jax.experimental import pallas as pl
from jax.experimental.pallas import tpu as pltpu
from jax.experimental.pallas import tpu_sc as plsc

NC = 2
NS = 16
NW = NC * NS
CHUNK = 128


def _make_deg_kernel(n2, ept2):
    mesh = plsc.VectorSubcoreMesh(core_axis_name="c", subcore_axis_name="s")

    @functools.partial(
        pl.kernel,
        out_type=jax.ShapeDtypeStruct((NC, NS, 1, n2), jnp.float32),
        mesh=mesh,
        scratch_types=[
            pltpu.VMEM((ept2,), jnp.int32),
            pltpu.VMEM((n2,), jnp.float32),
        ],
        compiler_params=pltpu.CompilerParams(needs_layout_passes=False),
    )
    def deg_kernel(idx_hbm, zeros_hbm, out_hbm, idx_v, hist):
        cid = lax.axis_index("c")
        sid = lax.axis_index("s")
        wid = sid * NC + cid
        pltpu.sync_copy(idx_hbm.at[wid, 0], idx_v)
        pltpu.sync_copy(zeros_hbm, hist)
        one16 = jnp.ones((16,), jnp.float32)

        def body(i, carry):
            vec = idx_v[pl.ds(pl.multiple_of(i * 16, 16), 16)]
            plsc.addupdate_scatter(hist, [vec], one16)
            return carry

        lax.fori_loop(0, ept2 // 16, body, 0)
        pltpu.sync_copy(hist, out_hbm.at[cid, sid, 0])

    return deg_kernel


def _make_agg_kernel(n, d, nch):
    stripe = n // NS
    mesh = plsc.VectorSubcoreMesh(core_axis_name="c", subcore_axis_name="s")

    @functools.partial(
        pl.kernel,
        out_type=jax.ShapeDtypeStruct((NC, n, d), jnp.float32),
        mesh=mesh,
        scratch_types=[
            pltpu.VMEM((nch, CHUNK), jnp.int32),
            pltpu.VMEM((nch, CHUNK), jnp.int32),
            pltpu.VMEM((CHUNK, d), jnp.float32),
            pltpu.VMEM_SHARED((n, d), jnp.float32),
            pltpu.SemaphoreType.DMA,
        ],
    )
    def agg_kernel(h_hbm, src_hbm, dst_hbm, zeros_hbm, out_hbm,
                   sidx, didx, rows, acc, sem):
        cid = lax.axis_index("c")
        sid = lax.axis_index("s")
        wid = sid * NC + cid
        pltpu.sync_copy(zeros_hbm, acc.at[pl.ds(sid * stripe, stripe)])
        pltpu.sync_copy(src_hbm.at[wid], sidx)
        pltpu.sync_copy(dst_hbm.at[wid], didx)
        plsc.subcore_barrier()

        def body(c, carry):
            pltpu.async_copy(h_hbm.at[sidx.at[c]], rows, sem).wait()
            pltpu.sync_copy(rows, acc.at[didx.at[c]], add=True)
            return carry

        lax.fori_loop(0, nch, body, 0)
        plsc.subcore_barrier()
        pltpu.sync_copy(
            acc.at[pl.ds(sid * stripe, stripe)],
            out_hbm.at[cid, pl.ds(sid * stripe, stripe)],
        )

    return agg_kernel


def _mm1_body(x_ref, w_ref, ds_ref, dd_ref, h_ref, ns_ref, nd_ref):
    ns = lax.rsqrt(jnp.maximum(
        jnp.sum(ds_ref[...], axis=1, keepdims=True), 1.0))
    nd = lax.rsqrt(jnp.maximum(
        jnp.sum(dd_ref[...], axis=1, keepdims=True), 1.0))
    h_ref[...] = jnp.dot(x_ref[...], w_ref[...],
                         preferred_element_type=jnp.float32) * ns
    ns_ref[...] = ns
    nd_ref[...] = nd


def _mid_body(p0, p1, nd, ns, b, o_ref):
    t = (p0[...] + p1[...]) * nd[...] + b[...]
    o_ref[...] = jnp.maximum(t, 0.0) * ns[...]


def _fin_body(p0, p1, nd, w, b, o_ref):
    s = p0[...] + p1[...]
    o_ref[...] = jnp.dot(s, w[...], preferred_element_type=jnp.float32) * nd[...] + b[...]


def kernel(x, edge_index, W1, b1, W2, b2):
    n, d_in = x.shape
    d_hid = W1.shape[1]
    n_cls = W2.shape[1]
    e = edge_index.shape[1]
    assert (2 * e) % (NW * 16) == 0 and n % NS == 0

    npad = -(-n // (8 * NS)) * (8 * NS)
    n2pad = -(-(2 * n) // 16) * 16
    src = edge_index[0]
    dst = edge_index[1]
    epad = -(-e // (2 * NW * CHUNK)) * (2 * NW * CHUNK)
    ept = epad // NW
    nch = ept // CHUNK
    srcp = jnp.concatenate([src, jnp.zeros((epad - e,), jnp.int32)])
    dstp = jnp.concatenate([dst, jnp.full((epad - e,), n, jnp.int32)])
    srcr = srcp.reshape(NW, nch, CHUNK)
    dstr = dstp.reshape(NW, nch, CHUNK)
    degidx = jnp.concatenate([src, dst + n]).reshape(NW, 1, 2 * e // NW)

    zeros_h = jnp.zeros((npad // NS, d_hid), jnp.float32)

    degpart = _make_deg_kernel(n2pad, 2 * e // NW)(
        degidx, jnp.zeros((n2pad,), jnp.float32))
    deg_t = degpart.reshape(NW, n2pad).T

    bn = 1000
    noff = n // bn
    grid = (n // bn,)
    h1p, ns_col, nd_col = pl.pallas_call(
        _mm1_body,
        grid=grid,
        in_specs=[
            pl.BlockSpec((bn, d_in), lambda i: (i, 0)),
            pl.BlockSpec((d_in, d_hid), lambda i: (0, 0)),
            pl.BlockSpec((bn, NW), lambda i: (i, 0)),
            pl.BlockSpec((bn, NW), lambda i: (i + noff, 0)),
        ],
        out_specs=[
            pl.BlockSpec((bn, d_hid), lambda i: (i, 0)),
            pl.BlockSpec((bn, 1), lambda i: (i, 0)),
            pl.BlockSpec((bn, 1), lambda i: (i, 0)),
        ],
        out_shape=[
            jax.ShapeDtypeStruct((n, d_hid), jnp.float32),
            jax.ShapeDtypeStruct((n, 1), jnp.float32),
            jax.ShapeDtypeStruct((n, 1), jnp.float32),
        ],
    )(x, W1, deg_t, deg_t)

    agg_fn = _make_agg_kernel(npad, d_hid, nch)
    part1 = agg_fn(h1p, srcr, dstr, zeros_h)

    h2p = pl.pallas_call(
        _mid_body,
        grid=grid,
        in_specs=[
            pl.BlockSpec((bn, d_hid), lambda i: (i, 0)),
            pl.BlockSpec((bn, d_hid), lambda i: (i, 0)),
            pl.BlockSpec((bn, 1), lambda i: (i, 0)),
            pl.BlockSpec((bn, 1), lambda i: (i, 0)),
            pl.BlockSpec((1, d_hid), lambda i: (0, 0)),
        ],
        out_specs=pl.BlockSpec((bn, d_hid), lambda i: (i, 0)),
        out_shape=jax.ShapeDtypeStruct((n, d_hid), jnp.float32),
    )(part1[0, :n], part1[1, :n], nd_col, ns_col, b1.reshape(1, d_hid))

    part2 = agg_fn(h2p, srcr, dstr, zeros_h)

    out = pl.pallas_call(
        _fin_body,
        grid=grid,
        in_specs=[
            pl.BlockSpec((bn, d_hid), lambda i: (i, 0)),
            pl.BlockSpec((bn, d_hid), lambda i: (i, 0)),
            pl.BlockSpec((bn, 1), lambda i: (i, 0)),
            pl.BlockSpec((d_hid, n_cls), lambda i: (0, 0)),
            pl.BlockSpec((1, n_cls), lambda i: (0, 0)),
        ],
        out_specs=pl.BlockSpec((bn, n_cls), lambda i: (i, 0)),
        out_shape=jax.ShapeDtypeStruct((n, n_cls), jnp.float32),
    )(part2[0, :n], part2[1, :n], nd_col, W2, b2.reshape(1, n_cls))

    return out


# speedup vs baseline: 2.5206x; 2.5206x over previous
import functools

import jax
import jax.numpy as jnp
from jax import lax
from jax.experimental import pallas as pl
from jax.experimental.pallas import tpu as pltpu
from jax.experimental.pallas import tpu_sc as plsc

NC = 2
NS = 16
NW = NC * NS
CHUNK = 80


def _make_deg_kernel(n2, ept2):
    mesh = plsc.VectorSubcoreMesh(core_axis_name="c", subcore_axis_name="s")

    @functools.partial(
        pl.kernel,
        out_type=jax.ShapeDtypeStruct((NC, NS, 1, n2), jnp.float32),
        mesh=mesh,
        scratch_types=[
            pltpu.VMEM((ept2,), jnp.int32),
            pltpu.VMEM((n2,), jnp.float32),
        ],
        compiler_params=pltpu.CompilerParams(needs_layout_passes=False),
    )
    def deg_kernel(idx_hbm, zeros_hbm, out_hbm, idx_v, hist):
        cid = lax.axis_index("c")
        sid = lax.axis_index("s")
        wid = sid * NC + cid
        pltpu.sync_copy(idx_hbm.at[wid, 0], idx_v)
        pltpu.sync_copy(zeros_hbm, hist)
        one16 = jnp.ones((16,), jnp.float32)

        def body(i, carry):
            vec = idx_v[pl.ds(pl.multiple_of(i * 16, 16), 16)]
            plsc.addupdate_scatter(hist, [vec], one16)
            return carry

        lax.fori_loop(0, ept2 // 16, body, 0)
        pltpu.sync_copy(hist, out_hbm.at[cid, sid, 0])

    return deg_kernel


def _make_agg_kernel(n, d, nch):
    stripe = n // NS
    mesh = plsc.VectorSubcoreMesh(core_axis_name="c", subcore_axis_name="s")

    @functools.partial(
        pl.kernel,
        out_type=jax.ShapeDtypeStruct((NC, n, d), jnp.float32),
        mesh=mesh,
        scratch_types=[
            pltpu.VMEM((nch, CHUNK), jnp.int32),
            pltpu.VMEM((CHUNK,), jnp.int32),
            pltpu.VMEM((CHUNK,), jnp.int32),
            pltpu.VMEM((CHUNK, d), jnp.float32),
            pltpu.VMEM((CHUNK, d), jnp.float32),
            pltpu.VMEM_SHARED((n, d), jnp.float32),
            pltpu.SemaphoreType.DMA,
            pltpu.SemaphoreType.DMA,
            pltpu.SemaphoreType.DMA,
            pltpu.SemaphoreType.DMA,
        ],
    )
    def agg_kernel(h_hbm, src_hbm, dst_hbm, zeros_hbm, out_hbm,
                   sidx, didx0, didx1, rows0, rows1, acc,
                   sem0, sem1, dsem0, dsem1):
        cid = lax.axis_index("c")
        sid = lax.axis_index("s")
        wid = sid * NC + cid
        pltpu.sync_copy(zeros_hbm, acc.at[pl.ds(sid * stripe, stripe)])
        pltpu.sync_copy(src_hbm.at[wid], sidx)
        plsc.subcore_barrier()

        base = wid * nch
        pltpu.async_copy(h_hbm.at[sidx.at[0]], rows0, sem0)
        pltpu.async_copy(dst_hbm.at[base, 0], didx0, dsem0)

        def outer(o, carry):
            c0 = o * 2
            c1 = c0 + 1
            pltpu.make_async_copy(h_hbm.at[sidx.at[0]], rows0, sem0).wait()
            pltpu.async_copy(h_hbm.at[sidx.at[c1]], rows1, sem1)
            pltpu.async_copy(dst_hbm.at[base + c1, 0], didx1, dsem1)
            pltpu.make_async_copy(dst_hbm.at[base, 0], didx0, dsem0).wait()
            pltpu.sync_copy(rows0, acc.at[didx0], add=True)

            pltpu.make_async_copy(h_hbm.at[sidx.at[0]], rows1, sem1).wait()
            pltpu.async_copy(h_hbm.at[sidx.at[c1 + 1]], rows0, sem0)
            pltpu.async_copy(dst_hbm.at[base + c1 + 1, 0], didx0, dsem0)
            pltpu.make_async_copy(dst_hbm.at[base, 0], didx1, dsem1).wait()
            pltpu.sync_copy(rows1, acc.at[didx1], add=True)
            return carry

        lax.fori_loop(0, nch // 2, outer, 0)
        pltpu.make_async_copy(h_hbm.at[sidx.at[0]], rows0, sem0).wait()
        pltpu.make_async_copy(dst_hbm.at[base, 0], didx0, dsem0).wait()
        pltpu.sync_copy(rows0, acc.at[didx0], add=True)
        plsc.subcore_barrier()
        pltpu.sync_copy(
            acc.at[pl.ds(sid * stripe, stripe)],
            out_hbm.at[cid, pl.ds(sid * stripe, stripe)],
        )

    return agg_kernel


def _mm1_body(x_ref, w_ref, ds_ref, dd_ref, h_ref, ns_ref, nd_ref):
    ns = lax.rsqrt(jnp.maximum(
        jnp.sum(ds_ref[...], axis=1, keepdims=True), 1.0))
    nd = lax.rsqrt(jnp.maximum(
        jnp.sum(dd_ref[...], axis=1, keepdims=True), 1.0))
    h_ref[...] = jnp.dot(x_ref[...], w_ref[...],
                         preferred_element_type=jnp.float32) * ns
    ns_ref[...] = ns
    nd_ref[...] = nd


def _mid_body(p0, p1, nd, ns, b, o_ref):
    t = (p0[...] + p1[...]) * nd[...] + b[...]
    o_ref[...] = jnp.maximum(t, 0.0) * ns[...]


def _fin_body(p0, p1, nd, w, b, o_ref):
    s = p0[...] + p1[...]
    o_ref[...] = jnp.dot(s, w[...], preferred_element_type=jnp.float32) * nd[...] + b[...]


def kernel(x, edge_index, W1, b1, W2, b2):
    n, d_in = x.shape
    d_hid = W1.shape[1]
    n_cls = W2.shape[1]
    e = edge_index.shape[1]
    assert (2 * e) % (NW * 16) == 0 and n % NS == 0

    npad = -(-n // (8 * NS)) * (8 * NS)
    n2pad = -(-(2 * n) // 16) * 16
    src = edge_index[0]
    dst = edge_index[1]
    epad = -(-e // (NW * CHUNK)) * (NW * CHUNK)
    if (epad // (NW * CHUNK)) % 2 == 0:
        epad += NW * CHUNK
    ept = epad // NW
    nch = ept // CHUNK
    srcp = jnp.concatenate([src, jnp.zeros((epad - e,), jnp.int32)])
    dstp = jnp.concatenate([dst, jnp.full((epad - e,), n, jnp.int32)])
    srcr = srcp.reshape(NW, nch, CHUNK)
    dstr = dstp.reshape(NW * nch, 1, CHUNK)
    degidx = jnp.concatenate([src, dst + n]).reshape(NW, 1, 2 * e // NW)

    zeros_h = jnp.zeros((npad // NS, d_hid), jnp.float32)

    degpart = _make_deg_kernel(n2pad, 2 * e // NW)(
        degidx, jnp.zeros((n2pad,), jnp.float32))
    deg_t = degpart.reshape(NW, n2pad).T

    bn = 1000
    noff = n // bn
    grid = (n // bn,)
    h1p, ns_col, nd_col = pl.pallas_call(
        _mm1_body,
        grid=grid,
        in_specs=[
            pl.BlockSpec((bn, d_in), lambda i: (i, 0)),
            pl.BlockSpec((d_in, d_hid), lambda i: (0, 0)),
            pl.BlockSpec((bn, NW), lambda i: (i, 0)),
            pl.BlockSpec((bn, NW), lambda i: (i + noff, 0)),
        ],
        out_specs=[
            pl.BlockSpec((bn, d_hid), lambda i: (i, 0)),
            pl.BlockSpec((bn, 1), lambda i: (i, 0)),
            pl.BlockSpec((bn, 1), lambda i: (i, 0)),
        ],
        out_shape=[
            jax.ShapeDtypeStruct((n, d_hid), jnp.float32),
            jax.ShapeDtypeStruct((n, 1), jnp.float32),
            jax.ShapeDtypeStruct((n, 1), jnp.float32),
        ],
    )(x, W1, deg_t, deg_t)

    agg_fn = _make_agg_kernel(npad, d_hid, nch)
    part1 = agg_fn(h1p, srcr, dstr, zeros_h)

    h2p = pl.pallas_call(
        _mid_body,
        grid=grid,
        in_specs=[
            pl.BlockSpec((bn, d_hid), lambda i: (i, 0)),
            pl.BlockSpec((bn, d_hid), lambda i: (i, 0)),
            pl.BlockSpec((bn, 1), lambda i: (i, 0)),
            pl.BlockSpec((bn, 1), lambda i: (i, 0)),
            pl.BlockSpec((1, d_hid), lambda i: (0, 0)),
        ],
        out_specs=pl.BlockSpec((bn, d_hid), lambda i: (i, 0)),
        out_shape=jax.ShapeDtypeStruct((n, d_hid), jnp.float32),
    )(part1[0, :n], part1[1, :n], nd_col, ns_col, b1.reshape(1, d_hid))

    part2 = agg_fn(h2p, srcr, dstr, zeros_h)

    out = pl.pallas_call(
        _fin_body,
        grid=grid,
        in_specs=[
            pl.BlockSpec((bn, d_hid), lambda i: (i, 0)),
            pl.BlockSpec((bn, d_hid), lambda i: (i, 0)),
            pl.BlockSpec((bn, 1), lambda i: (i, 0)),
            pl.BlockSpec((d_hid, n_cls), lambda i: (0, 0)),
            pl.BlockSpec((1, n_cls), lambda i: (0, 0)),
        ],
        out_specs=pl.BlockSpec((bn, n_cls), lambda i: (i, 0)),
        out_shape=jax.ShapeDtypeStruct((n, n_cls), jnp.float32),
    )(part2[0, :n], part2[1, :n], nd_col, W2, b2.reshape(1, n_cls))

    return out


# speedup vs baseline: 2.6783x; 1.0626x over previous
import functools

import jax
import jax.numpy as jnp
from jax import lax
from jax.experimental import pallas as pl
from jax.experimental.pallas import tpu as pltpu
from jax.experimental.pallas import tpu_sc as plsc

NC = 2
NS = 16
NW = NC * NS
CHUNK = 80


def _make_deg_kernel(n2, ept2):
    mesh = plsc.VectorSubcoreMesh(core_axis_name="c", subcore_axis_name="s")

    @functools.partial(
        pl.kernel,
        out_type=jax.ShapeDtypeStruct((NC, NS, 1, n2), jnp.float32),
        mesh=mesh,
        scratch_types=[
            pltpu.VMEM((ept2,), jnp.int32),
            pltpu.VMEM((n2,), jnp.float32),
        ],
        compiler_params=pltpu.CompilerParams(needs_layout_passes=False),
    )
    def deg_kernel(edge_hbm, zeros_hbm, out_hbm, idx_v, hist):
        cid = lax.axis_index("c")
        sid = lax.axis_index("s")
        wid = sid * NC + cid
        nbin = jnp.full((16,), n2 // 2, jnp.int32)
        pltpu.sync_copy(zeros_hbm, hist)
        one16 = jnp.ones((16,), jnp.float32)

        pltpu.sync_copy(edge_hbm.at[0, wid, 0], idx_v)

        def body(i, carry):
            vec = idx_v[pl.ds(pl.multiple_of(i * 16, 16), 16)]
            plsc.addupdate_scatter(hist, [vec], one16)
            return carry

        lax.fori_loop(0, ept2 // 16, body, 0)

        pltpu.sync_copy(edge_hbm.at[1, wid, 0], idx_v)

        def body2(i, carry):
            vec = idx_v[pl.ds(pl.multiple_of(i * 16, 16), 16)] + nbin
            plsc.addupdate_scatter(hist, [vec], one16)
            return carry

        lax.fori_loop(0, ept2 // 16, body2, 0)
        pltpu.sync_copy(hist, out_hbm.at[cid, sid, 0])

    return deg_kernel


def _make_agg_kernel(n, d, nch):
    stripe = n // NS
    mesh = plsc.VectorSubcoreMesh(core_axis_name="c", subcore_axis_name="s")

    @functools.partial(
        pl.kernel,
        out_type=jax.ShapeDtypeStruct((NC, n, d), jnp.float32),
        mesh=mesh,
        scratch_types=[
            pltpu.VMEM((nch * CHUNK,), jnp.int32),
            pltpu.VMEM((CHUNK,), jnp.int32),
            pltpu.VMEM((CHUNK,), jnp.int32),
            pltpu.VMEM((CHUNK, d), jnp.float32),
            pltpu.VMEM((CHUNK, d), jnp.float32),
            pltpu.VMEM_SHARED((n, d), jnp.float32),
            pltpu.SemaphoreType.DMA,
            pltpu.SemaphoreType.DMA,
            pltpu.SemaphoreType.DMA,
            pltpu.SemaphoreType.DMA,
        ],
    )
    def agg_kernel(h_hbm, src_hbm, dst_hbm, zeros_hbm, out_hbm,
                   sidx, didx0, didx1, rows0, rows1, acc,
                   sem0, sem1, dsem0, dsem1):
        cid = lax.axis_index("c")
        sid = lax.axis_index("s")
        wid = sid * NC + cid
        pltpu.sync_copy(zeros_hbm, acc.at[pl.ds(sid * stripe, stripe)])
        pltpu.sync_copy(src_hbm.at[0, wid, 0], sidx)
        plsc.subcore_barrier()

        def src_at(c):
            return sidx.at[pl.ds(pl.multiple_of(c * CHUNK, 16), CHUNK)]

        base = wid * nch
        pltpu.async_copy(h_hbm.at[src_at(0)], rows0, sem0)
        pltpu.async_copy(dst_hbm.at[1, base, 0], didx0, dsem0)

        def outer(o, carry):
            c0 = o * 2
            c1 = c0 + 1
            pltpu.make_async_copy(h_hbm.at[src_at(0)], rows0, sem0).wait()
            pltpu.async_copy(h_hbm.at[src_at(c1)], rows1, sem1)
            pltpu.async_copy(dst_hbm.at[1, base + c1, 0], didx1, dsem1)
            pltpu.make_async_copy(dst_hbm.at[1, base, 0], didx0, dsem0).wait()
            pltpu.sync_copy(rows0, acc.at[didx0], add=True)

            pltpu.make_async_copy(h_hbm.at[src_at(0)], rows1, sem1).wait()
            pltpu.async_copy(h_hbm.at[src_at(c1 + 1)], rows0, sem0)
            pltpu.async_copy(dst_hbm.at[1, base + c1 + 1, 0], didx0, dsem0)
            pltpu.make_async_copy(dst_hbm.at[1, base, 0], didx1, dsem1).wait()
            pltpu.sync_copy(rows1, acc.at[didx1], add=True)
            return carry

        lax.fori_loop(0, nch // 2, outer, 0)
        pltpu.make_async_copy(h_hbm.at[src_at(0)], rows0, sem0).wait()
        pltpu.make_async_copy(dst_hbm.at[1, base, 0], didx0, dsem0).wait()
        pltpu.sync_copy(rows0, acc.at[didx0], add=True)
        plsc.subcore_barrier()
        pltpu.sync_copy(
            acc.at[pl.ds(sid * stripe, stripe)],
            out_hbm.at[cid, pl.ds(sid * stripe, stripe)],
        )

    return agg_kernel


def _mm1_body(x_ref, w_ref, ds_ref, dd_ref, h_ref, ns_ref, nd_ref):
    ns = lax.rsqrt(jnp.maximum(
        jnp.sum(ds_ref[...], axis=1, keepdims=True), 1.0))
    nd = lax.rsqrt(jnp.maximum(
        jnp.sum(dd_ref[...], axis=1, keepdims=True), 1.0))
    h_ref[...] = jnp.dot(x_ref[...], w_ref[...],
                         preferred_element_type=jnp.float32) * ns
    ns_ref[...] = ns
    nd_ref[...] = nd


def _mid_body(p0, p1, nd, ns, b, o_ref):
    t = (p0[0] + p1[0]) * nd[...] + b[...]
    o_ref[...] = jnp.maximum(t, 0.0) * ns[...]


def _fin_body(p0, p1, nd, w, b, o_ref):
    s = p0[0] + p1[0]
    o_ref[...] = jnp.dot(s, w[...], preferred_element_type=jnp.float32) * nd[...] + b[...]


def kernel(x, edge_index, W1, b1, W2, b2):
    n, d_in = x.shape
    d_hid = W1.shape[1]
    n_cls = W2.shape[1]
    e = edge_index.shape[1]
    assert (2 * e) % (NW * 16) == 0 and n % NS == 0

    npad = -(-n // (8 * NS)) * (8 * NS)
    n2pad = 2 * n
    epad = -(-e // (NW * CHUNK)) * (NW * CHUNK)
    if (epad // (NW * CHUNK)) % 2 == 0:
        epad += NW * CHUNK
    ept = epad // NW
    nch = ept // CHUNK
    if epad == e:
        edges = edge_index
    else:
        pad = jnp.stack([
            jnp.zeros((epad - e,), jnp.int32),
            jnp.full((epad - e,), n, jnp.int32),
        ])
        edges = jnp.concatenate([edge_index, pad], axis=1)
    srcr = edges.reshape(2, NW, 1, ept)
    dstr = edges.reshape(2, NW * nch, 1, CHUNK)

    zeros_h = jnp.zeros((npad // NS, d_hid), jnp.float32)

    degpart = _make_deg_kernel(n2pad, e // NW)(
        edge_index.reshape(2, NW, 1, e // NW),
        jnp.zeros((n2pad,), jnp.float32))
    deg_t = degpart.reshape(NW, n2pad).T

    bn = 1000
    noff = n // bn
    grid = (n // bn,)
    h1p, ns_col, nd_col = pl.pallas_call(
        _mm1_body,
        grid=grid,
        in_specs=[
            pl.BlockSpec((bn, d_in), lambda i: (i, 0)),
            pl.BlockSpec((d_in, d_hid), lambda i: (0, 0)),
            pl.BlockSpec((bn, NW), lambda i: (i, 0)),
            pl.BlockSpec((bn, NW), lambda i: (i + noff, 0)),
        ],
        out_specs=[
            pl.BlockSpec((bn, d_hid), lambda i: (i, 0)),
            pl.BlockSpec((bn, 1), lambda i: (i, 0)),
            pl.BlockSpec((bn, 1), lambda i: (i, 0)),
        ],
        out_shape=[
            jax.ShapeDtypeStruct((n, d_hid), jnp.float32),
            jax.ShapeDtypeStruct((n, 1), jnp.float32),
            jax.ShapeDtypeStruct((n, 1), jnp.float32),
        ],
    )(x, W1, deg_t, deg_t)

    agg_fn = _make_agg_kernel(npad, d_hid, nch)
    part1 = agg_fn(h1p, srcr, dstr, zeros_h)

    h2p = pl.pallas_call(
        _mid_body,
        grid=grid,
        in_specs=[
            pl.BlockSpec((1, bn, d_hid), lambda i: (0, i, 0)),
            pl.BlockSpec((1, bn, d_hid), lambda i: (1, i, 0)),
            pl.BlockSpec((bn, 1), lambda i: (i, 0)),
            pl.BlockSpec((bn, 1), lambda i: (i, 0)),
            pl.BlockSpec((1, d_hid), lambda i: (0, 0)),
        ],
        out_specs=pl.BlockSpec((bn, d_hid), lambda i: (i, 0)),
        out_shape=jax.ShapeDtypeStruct((n, d_hid), jnp.float32),
    )(part1, part1, nd_col, ns_col, b1.reshape(1, d_hid))

    part2 = agg_fn(h2p, srcr, dstr, zeros_h)

    out = pl.pallas_call(
        _fin_body,
        grid=grid,
        in_specs=[
            pl.BlockSpec((1, bn, d_hid), lambda i: (0, i, 0)),
            pl.BlockSpec((1, bn, d_hid), lambda i: (1, i, 0)),
            pl.BlockSpec((bn, 1), lambda i: (i, 0)),
            pl.BlockSpec((d_hid, n_cls), lambda i: (0, 0)),
            pl.BlockSpec((1, n_cls), lambda i: (0, 0)),
        ],
        out_specs=pl.BlockSpec((bn, n_cls), lambda i: (i, 0)),
        out_shape=jax.ShapeDtypeStruct((n, n_cls), jnp.float32),
    )(part2, part2, nd_col, W2, b2.reshape(1, n_cls))

    return out


# speedup vs baseline: 3.6687x; 1.3698x over previous
import functools

import jax
import jax.numpy as jnp
from jax import lax
from jax.experimental import pallas as pl
from jax.experimental.pallas import tpu as pltpu
from jax.experimental.pallas import tpu_sc as plsc

NC = 2
NS = 16
NW = NC * NS
CHUNK = 80


def _make_deg_kernel(n2, ept2):
    mesh = plsc.VectorSubcoreMesh(core_axis_name="c", subcore_axis_name="s")

    @functools.partial(
        pl.kernel,
        out_type=jax.ShapeDtypeStruct((NC, NS, 1, n2), jnp.float32),
        mesh=mesh,
        scratch_types=[
            pltpu.VMEM((ept2,), jnp.int32),
            pltpu.VMEM((n2,), jnp.float32),
        ],
        compiler_params=pltpu.CompilerParams(needs_layout_passes=False),
    )
    def deg_kernel(edge_hbm, zeros_hbm, out_hbm, idx_v, hist):
        cid = lax.axis_index("c")
        sid = lax.axis_index("s")
        wid = sid * NC + cid
        nbin = jnp.full((16,), n2 // 2, jnp.int32)
        pltpu.sync_copy(zeros_hbm, hist)
        one16 = jnp.ones((16,), jnp.float32)

        pltpu.sync_copy(edge_hbm.at[0, wid, 0], idx_v)

        def body(i, carry):
            vec = idx_v[pl.ds(pl.multiple_of(i * 16, 16), 16)]
            plsc.addupdate_scatter(hist, [vec], one16)
            return carry

        lax.fori_loop(0, ept2 // 16, body, 0)

        pltpu.sync_copy(edge_hbm.at[1, wid, 0], idx_v)

        def body2(i, carry):
            vec = idx_v[pl.ds(pl.multiple_of(i * 16, 16), 16)] + nbin
            plsc.addupdate_scatter(hist, [vec], one16)
            return carry

        lax.fori_loop(0, ept2 // 16, body2, 0)
        pltpu.sync_copy(hist, out_hbm.at[cid, sid, 0])

    return deg_kernel


def _make_agg_kernel(n, d, nch):
    stripe = n // NS
    mesh = plsc.VectorSubcoreMesh(core_axis_name="c", subcore_axis_name="s")

    @functools.partial(
        pl.kernel,
        out_type=jax.ShapeDtypeStruct((NC, n, d), jnp.float32),
        mesh=mesh,
        scratch_types=[
            pltpu.VMEM((nch * CHUNK,), jnp.int32),
            [pltpu.VMEM((CHUNK,), jnp.int32)] * 3,
            [pltpu.VMEM((CHUNK, d), jnp.float32)] * 3,
            [pltpu.SemaphoreType.DMA] * 3,
            [pltpu.SemaphoreType.DMA] * 3,
            [pltpu.SemaphoreType.DMA] * 3,
            pltpu.VMEM_SHARED((n, d), jnp.float32),
        ],
    )
    def agg_kernel(h_hbm, src_hbm, dst_hbm, zeros_hbm, out_hbm,
                   sidx, didx, rows, gsem, dsem, ssem, acc):
        cid = lax.axis_index("c")
        sid = lax.axis_index("s")
        wid = sid * NC + cid
        pltpu.sync_copy(zeros_hbm, acc.at[pl.ds(sid * stripe, stripe)])
        pltpu.sync_copy(src_hbm.at[0, wid, 0], sidx)
        plsc.subcore_barrier()

        base = wid * nch

        def src_at(c):
            return sidx.at[pl.ds(pl.multiple_of(c * CHUNK, 16), CHUNK)]

        def issue(c, j):
            pltpu.async_copy(dst_hbm.at[1, base + c, 0], didx[j], dsem[j])
            pltpu.async_copy(h_hbm.at[src_at(c)], rows[j], gsem[j])

        def gwait(j):
            pltpu.make_async_copy(h_hbm.at[src_at(0)], rows[j], gsem[j]).wait()
            pltpu.make_async_copy(dst_hbm.at[1, base, 0], didx[j], dsem[j]).wait()

        def scat(j):
            pltpu.async_copy(rows[j], acc.at[didx[j]], ssem[j], add=True)

        def swait(j):
            pltpu.make_async_copy(rows[j], acc.at[didx[j]], ssem[j]).wait()

        issue(0, 0)
        issue(1, 1)
        gwait(0)
        scat(0)
        issue(2, 2)
        gwait(1)
        scat(1)

        def steady(o, carry):
            s0 = 2 + o * 3

            def step(s, j, jn):
                swait(jn)
                issue(s + 1, jn)
                gwait(j)
                scat(j)

            step(s0, 2, 0)
            step(s0 + 1, 0, 1)
            step(s0 + 2, 1, 2)
            return carry

        lax.fori_loop(0, (nch - 5) // 3, steady, 0)
        j_a = (nch - 3) % 3
        j_b = (nch - 2) % 3
        j_c = (nch - 1) % 3
        swait(j_b)
        issue(nch - 2, j_b)
        gwait(j_a)
        scat(j_a)
        swait(j_c)
        issue(nch - 1, j_c)
        gwait(j_b)
        scat(j_b)
        swait(j_a)
        gwait(j_c)
        scat(j_c)
        swait(j_b)
        swait(j_c)
        plsc.subcore_barrier()
        pltpu.sync_copy(
            acc.at[pl.ds(sid * stripe, stripe)],
            out_hbm.at[cid, pl.ds(sid * stripe, stripe)],
        )

    return agg_kernel


def _mm1_body(x_ref, w_ref, ds_ref, dd_ref, h_ref, ns_ref, nd_ref):
    ns = lax.rsqrt(jnp.maximum(
        jnp.sum(ds_ref[...], axis=1, keepdims=True), 1.0))
    nd = lax.rsqrt(jnp.maximum(
        jnp.sum(dd_ref[...], axis=1, keepdims=True), 1.0))
    h_ref[...] = jnp.dot(x_ref[...], w_ref[...],
                         preferred_element_type=jnp.float32) * ns
    ns_ref[...] = ns
    nd_ref[...] = nd


def _mid_body(p0, p1, nd, ns, b, o_ref):
    t = (p0[0] + p1[0]) * nd[...] + b[...]
    o_ref[...] = jnp.maximum(t, 0.0) * ns[...]


def _fin_body(p0, p1, nd, w, b, o_ref):
    s = p0[0] + p1[0]
    o_ref[...] = jnp.dot(s, w[...], preferred_element_type=jnp.float32) * nd[...] + b[...]


def kernel(x, edge_index, W1, b1, W2, b2):
    n, d_in = x.shape
    d_hid = W1.shape[1]
    n_cls = W2.shape[1]
    e = edge_index.shape[1]
    assert (2 * e) % (NW * 16) == 0 and n % NS == 0

    npad = -(-n // (8 * NS)) * (8 * NS)
    n2pad = 2 * n
    epad = -(-e // (NW * CHUNK)) * (NW * CHUNK)
    while (epad // (NW * CHUNK)) % 3 != 2 or epad // (NW * CHUNK) < 5:
        epad += NW * CHUNK
    ept = epad // NW
    nch = ept // CHUNK
    if epad == e:
        edges = edge_index
    else:
        pad = jnp.stack([
            jnp.zeros((epad - e,), jnp.int32),
            jnp.full((epad - e,), n, jnp.int32),
        ])
        edges = jnp.concatenate([edge_index, pad], axis=1)
    srcr = edges.reshape(2, NW, 1, ept)
    dstr = edges.reshape(2, NW * nch, 1, CHUNK)

    zeros_h = jnp.zeros((npad // NS, d_hid), jnp.float32)

    degpart = _make_deg_kernel(n2pad, e // NW)(
        edge_index.reshape(2, NW, 1, e // NW),
        jnp.zeros((n2pad,), jnp.float32))
    deg_t = degpart.reshape(NW, n2pad).T

    bn = 1000
    noff = n // bn
    grid = (n // bn,)
    h1p, ns_col, nd_col = pl.pallas_call(
        _mm1_body,
        grid=grid,
        in_specs=[
            pl.BlockSpec((bn, d_in), lambda i: (i, 0)),
            pl.BlockSpec((d_in, d_hid), lambda i: (0, 0)),
            pl.BlockSpec((bn, NW), lambda i: (i, 0)),
            pl.BlockSpec((bn, NW), lambda i: (i + noff, 0)),
        ],
        out_specs=[
            pl.BlockSpec((bn, d_hid), lambda i: (i, 0)),
            pl.BlockSpec((bn, 1), lambda i: (i, 0)),
            pl.BlockSpec((bn, 1), lambda i: (i, 0)),
        ],
        out_shape=[
            jax.ShapeDtypeStruct((n, d_hid), jnp.float32),
            jax.ShapeDtypeStruct((n, 1), jnp.float32),
            jax.ShapeDtypeStruct((n, 1), jnp.float32),
        ],
    )(x, W1, deg_t, deg_t)

    agg_fn = _make_agg_kernel(npad, d_hid, nch)
    part1 = agg_fn(h1p, srcr, dstr, zeros_h)

    h2p = pl.pallas_call(
        _mid_body,
        grid=grid,
        in_specs=[
            pl.BlockSpec((1, bn, d_hid), lambda i: (0, i, 0)),
            pl.BlockSpec((1, bn, d_hid), lambda i: (1, i, 0)),
            pl.BlockSpec((bn, 1), lambda i: (i, 0)),
            pl.BlockSpec((bn, 1), lambda i: (i, 0)),
            pl.BlockSpec((1, d_hid), lambda i: (0, 0)),
        ],
        out_specs=pl.BlockSpec((bn, d_hid), lambda i: (i, 0)),
        out_shape=jax.ShapeDtypeStruct((n, d_hid), jnp.float32),
    )(part1, part1, nd_col, ns_col, b1.reshape(1, d_hid))

    part2 = agg_fn(h2p, srcr, dstr, zeros_h)

    out = pl.pallas_call(
        _fin_body,
        grid=grid,
        in_specs=[
            pl.BlockSpec((1, bn, d_hid), lambda i: (0, i, 0)),
            pl.BlockSpec((1, bn, d_hid), lambda i: (1, i, 0)),
            pl.BlockSpec((bn, 1), lambda i: (i, 0)),
            pl.BlockSpec((d_hid, n_cls), lambda i: (0, 0)),
            pl.BlockSpec((1, n_cls), lambda i: (0, 0)),
        ],
        out_specs=pl.BlockSpec((bn, n_cls), lambda i: (i, 0)),
        out_shape=jax.ShapeDtypeStruct((n, n_cls), jnp.float32),
    )(part2, part2, nd_col, W2, b2.reshape(1, n_cls))

    return out
